# Initial kernel scaffold; baseline (speedup 1.0000x reference)
import functools

import jax
import jax.numpy as jnp
from jax import lax
from jax.experimental import pallas as pl
from jax.experimental.pallas import tpu as pltpu
from jax.experimental.pallas import tpu_sc as plsc

N = 10000
C = 128
NPAD = 10240
NC, NS, L = 2, 16, 16
E_ROWS = 2560
EPB = 16
DEG_ROWS = E_ROWS // NS
EDGE_ROWS = E_ROWS // (NC * NS)
NSL = NPAD // NS


def _tc_prep_body(x_ref, w_ref, wih_ref, bih_ref, bhh_ref, o_ref):
    w = w_ref[...]
    gates = lax.dot_general(w, wih_ref[...], (((1,), (1,)), ((), ())),
                            preferred_element_type=jnp.float32)
    gates = gates + bih_ref[...] + bhh_ref[...]
    i_g = gates[:, 0:C]
    g_g = gates[:, 2 * C:3 * C]
    o_g = gates[:, 3 * C:4 * C]
    c = jax.nn.sigmoid(i_g) * jnp.tanh(g_g)
    h = jax.nn.sigmoid(o_g) * jnp.tanh(c)
    o_ref[...] = jnp.dot(x_ref[...], h, preferred_element_type=jnp.float32)


def _tc_prep(x_pad, w, w_ih, b_ih2, b_hh2):
    return pl.pallas_call(
        _tc_prep_body,
        out_shape=jax.ShapeDtypeStruct((NPAD, C), jnp.float32),
    )(x_pad, w, w_ih, b_ih2, b_hh2)


def _tc_comb_body(p_ref, o_ref):
    o_ref[...] = p_ref[0:N] + p_ref[NPAD:NPAD + N]


def _tc_combine(partials):
    return pl.pallas_call(
        _tc_comb_body,
        out_shape=jax.ShapeDtypeStruct((N, C), jnp.float32),
    )(partials)


def _sc_body(srcR, dstR, ewR, xw_hbm, out_hbm,
             deg_sm, dinv_sm, acc_sm,
             deg_t, dinv_t, src_t, dst_t, ew_t, norm_t, rows_t):
    cid = lax.axis_index("c")
    sid = lax.axis_index("s")
    cmask = jnp.where(cid == 0, jnp.float32(1.0), jnp.float32(0.0))

    nbase = sid * NSL

    ones = jnp.full((L,), 1.0, jnp.float32)

    def fill1(k, _):
        deg_t[pl.ds(k * L, L)] = ones
        return 0
    lax.fori_loop(0, NSL // L, fill1, 0)
    pltpu.sync_copy(deg_t, deg_sm.at[pl.ds(nbase, NSL)])
    plsc.subcore_barrier()

    dbase = sid * DEG_ROWS

    def deg_chunk(ci, _):
        r0 = dbase + ci * EPB
        pltpu.sync_copy(dstR.at[pl.ds(r0, EPB)], dst_t)
        pltpu.sync_copy(ewR.at[pl.ds(r0, EPB)], ew_t)

        def deg_row(j, _):
            pltpu.sync_copy(ew_t.at[j], deg_sm.at[dst_t.at[j]], add=True)
            return 0
        lax.fori_loop(0, EPB, deg_row, 0)
        return 0
    lax.fori_loop(0, DEG_ROWS // EPB, deg_chunk, 0)
    plsc.subcore_barrier()

    pltpu.sync_copy(deg_sm.at[pl.ds(nbase, NSL)], deg_t)

    def newton(k, _):
        x = deg_t[pl.ds(k * L, L)]
        i = plsc.bitcast(x, jnp.int32)
        i = jnp.int32(0x5F3759DF) - lax.shift_right_logical(i, 1)
        y = plsc.bitcast(i, jnp.float32)
        y = y * (1.5 - 0.5 * x * y * y)
        y = y * (1.5 - 0.5 * x * y * y)
        y = y * (1.5 - 0.5 * x * y * y)
        y = y * (1.5 - 0.5 * x * y * y)
        deg_t[pl.ds(k * L, L)] = y
        return 0
    lax.fori_loop(0, NSL // L, newton, 0)
    pltpu.sync_copy(deg_t, dinv_sm.at[pl.ds(nbase, NSL)])
    plsc.subcore_barrier()
    pltpu.sync_copy(dinv_sm, dinv_t)

    def init_chunk(ci, _):
        r0 = nbase + ci * 128
        pltpu.sync_copy(xw_hbm.at[pl.ds(r0, 128)], rows_t)

        def init_row(e, _):
            d = dinv_t[r0 + e]
            s = d * d * cmask
            for j in range(C // L):
                rows_t[e, pl.ds(j * L, L)] = rows_t[e, pl.ds(j * L, L)] * s
            return 0
        lax.fori_loop(0, 128, init_row, 0)
        pltpu.sync_copy(rows_t, acc_sm.at[pl.ds(r0, 128)])
        return 0
    lax.fori_loop(0, NSL // 128, init_chunk, 0)
    plsc.subcore_barrier()

    ebase = cid * (E_ROWS // NC) + sid * EDGE_ROWS

    def edge_chunk(ci, _):
        r0 = ebase + ci * EPB
        pltpu.sync_copy(srcR.at[pl.ds(r0, EPB)], src_t)
        pltpu.sync_copy(dstR.at[pl.ds(r0, EPB)], dst_t)
        pltpu.sync_copy(ewR.at[pl.ds(r0, EPB)], ew_t)

        def group(j, _):
            pltpu.sync_copy(xw_hbm.at[src_t.at[j]], rows_t)
            for k in range(128 // L):
                sv = src_t[j, pl.ds(k * L, L)]
                dv = dst_t[j, pl.ds(k * L, L)]
                wv = ew_t[j, pl.ds(k * L, L)]
                nv = (plsc.load_gather(dinv_t, [sv])
                      * plsc.load_gather(dinv_t, [dv]) * wv)
                norm_t[pl.ds(k * L, L)] = nv

            def scale_row(e, _):
                s = norm_t[e]
                for j8 in range(C // L):
                    rows_t[e, pl.ds(j8 * L, L)] = (
                        rows_t[e, pl.ds(j8 * L, L)] * s)
                return 0
            lax.fori_loop(0, 128, scale_row, 0)
            pltpu.sync_copy(rows_t, acc_sm.at[dst_t.at[j]], add=True)
            return 0
        lax.fori_loop(0, EPB, group, 0)
        return 0
    lax.fori_loop(0, EDGE_ROWS // EPB, edge_chunk, 0)
    plsc.subcore_barrier()

    pltpu.sync_copy(acc_sm.at[pl.ds(nbase, NSL)],
                    out_hbm.at[pl.ds(cid * NPAD + nbase, NSL)])


def _sc_edge(srcR, dstR, ewR, xw):
    mesh = plsc.VectorSubcoreMesh(core_axis_name="c", subcore_axis_name="s",
                                  num_cores=NC, num_subcores=NS)
    f = functools.partial(
        pl.kernel,
        out_type=jax.ShapeDtypeStruct((2 * NPAD, C), jnp.float32),
        mesh=mesh,
        scratch_types=[
            pltpu.VMEM_SHARED((NPAD,), jnp.float32),
            pltpu.VMEM_SHARED((NPAD,), jnp.float32),
            pltpu.VMEM_SHARED((NPAD, C), jnp.float32),
            pltpu.VMEM((NSL,), jnp.float32),
            pltpu.VMEM((NPAD,), jnp.float32),
            pltpu.VMEM((EPB, 128), jnp.int32),
            pltpu.VMEM((EPB, 128), jnp.int32),
            pltpu.VMEM((EPB, 128), jnp.float32),
            pltpu.VMEM((128,), jnp.float32),
            pltpu.VMEM((128, C), jnp.float32),
        ],
    )(_sc_body)
    return f(srcR, dstR, ewR, xw)


def kernel(X, edge_index, edge_weight, W, W_ih, W_hh, b_ih, b_hh):
    del W_hh
    x_pad = jnp.zeros((NPAD, C), jnp.float32).at[:N].set(X)
    xw = _tc_prep(x_pad, W, W_ih,
                  b_ih.reshape(1, 4 * C), b_hh.reshape(1, 4 * C))

    epad = E_ROWS * 128
    e = edge_index.shape[1]
    src = jnp.zeros((epad,), jnp.int32).at[:e].set(
        edge_index[0].astype(jnp.int32))
    dst = jnp.zeros((epad,), jnp.int32).at[:e].set(
        edge_index[1].astype(jnp.int32))
    ew = jnp.zeros((epad,), jnp.float32).at[:e].set(edge_weight)
    partials = _sc_edge(src.reshape(E_ROWS, 128), dst.reshape(E_ROWS, 128),
                        ew.reshape(E_ROWS, 128), xw)
    return _tc_combine(partials)


# speedup vs baseline: 12.8520x; 12.8520x over previous
import functools

import jax
import jax.numpy as jnp
from jax import lax
from jax.experimental import pallas as pl
from jax.experimental.pallas import tpu as pltpu
from jax.experimental.pallas import tpu_sc as plsc

N = 10000
C = 128
NPAD = 10240
NC, NS, L = 2, 16, 16
E_ROWS = 2560
EPB = 16
EDGE_ROWS = E_ROWS // (NC * NS)
NSL = NPAD // NS


def _tc_prep_body(x_ref, w_ref, wih_ref, bih_ref, bhh_ref, degp_ref,
                  o_ref, dinv_ref):
    w = w_ref[...]
    gates = lax.dot_general(w, wih_ref[...], (((1,), (1,)), ((), ())),
                            preferred_element_type=jnp.float32)
    gates = gates + bih_ref[...] + bhh_ref[...]
    i_g = gates[:, 0:C]
    g_g = gates[:, 2 * C:3 * C]
    o_g = gates[:, 3 * C:4 * C]
    c = jax.nn.sigmoid(i_g) * jnp.tanh(g_g)
    h = jax.nn.sigmoid(o_g) * jnp.tanh(c)
    o_ref[...] = jnp.dot(x_ref[...], h, preferred_element_type=jnp.float32)
    deg = degp_ref[0:NPAD // C] + degp_ref[NPAD // C:] + 1.0
    dinv_ref[...] = jnp.where(deg > 0, lax.rsqrt(deg), 0.0)


def _tc_prep(x_pad, w, w_ih, b_ih2, b_hh2, degp):
    return pl.pallas_call(
        _tc_prep_body,
        out_shape=(jax.ShapeDtypeStruct((NPAD, C), jnp.float32),
                   jax.ShapeDtypeStruct((NPAD // C, C), jnp.float32)),
    )(x_pad, w, w_ih, b_ih2, b_hh2, degp)


def _tc_comb_body(p_ref, o_ref):
    o_ref[...] = p_ref[0:N] + p_ref[NPAD:NPAD + N]


def _tc_combine(partials):
    return pl.pallas_call(
        _tc_comb_body,
        out_shape=jax.ShapeDtypeStruct((N, C), jnp.float32),
    )(partials)


def _sc_deg_body(dstR, ewR, out_hbm, deg_sm, deg_t, dst_t, ew_t):
    cid = lax.axis_index("c")
    sid = lax.axis_index("s")
    nbase = sid * NSL

    zeros = jnp.zeros((L,), jnp.float32)

    def fill0(k, _):
        deg_t[pl.ds(k * L, L)] = zeros
        return 0
    lax.fori_loop(0, NSL // L, fill0, 0)
    pltpu.sync_copy(deg_t, deg_sm.at[pl.ds(nbase, NSL)])
    plsc.subcore_barrier()

    ebase = cid * (E_ROWS // NC) + sid * EDGE_ROWS

    def deg_chunk(ci, _):
        r0 = ebase + ci * EPB
        pltpu.sync_copy(dstR.at[pl.ds(r0, EPB)], dst_t)
        pltpu.sync_copy(ewR.at[pl.ds(r0, EPB)], ew_t)

        def deg_row(j, _):
            pltpu.sync_copy(ew_t.at[j], deg_sm.at[dst_t.at[j]], add=True)
            return 0
        lax.fori_loop(0, EPB, deg_row, 0)
        return 0
    lax.fori_loop(0, EDGE_ROWS // EPB, deg_chunk, 0)
    plsc.subcore_barrier()

    pltpu.sync_copy(deg_sm.at[pl.ds(nbase, NSL)],
                    out_hbm.at[pl.ds(cid * NPAD + nbase, NSL)])


def _sc_deg(dstR, ewR):
    mesh = plsc.VectorSubcoreMesh(core_axis_name="c", subcore_axis_name="s",
                                  num_cores=NC, num_subcores=NS)
    f = functools.partial(
        pl.kernel,
        out_type=jax.ShapeDtypeStruct((2 * NPAD,), jnp.float32),
        mesh=mesh,
        compiler_params=pltpu.CompilerParams(needs_layout_passes=False),
        scratch_types=[
            pltpu.VMEM_SHARED((NPAD,), jnp.float32),
            pltpu.VMEM((NSL,), jnp.float32),
            pltpu.VMEM((EPB, 128), jnp.int32),
            pltpu.VMEM((EPB, 128), jnp.float32),
        ],
    )(_sc_deg_body)
    return f(dstR, ewR)


def _sc_body(srcR, dstR, ewR, xw_hbm, dinv_hbm, out_hbm,
             acc_sm, dinv_t, src_t, dst_t, ew_t, norm_t, rows_t):
    cid = lax.axis_index("c")
    sid = lax.axis_index("s")
    cmask = jnp.where(cid == 0, jnp.float32(1.0), jnp.float32(0.0))
    nbase = sid * NSL

    pltpu.sync_copy(dinv_hbm, dinv_t)

    def init_chunk(ci, _):
        r0 = nbase + ci * 128
        pltpu.sync_copy(xw_hbm.at[pl.ds(r0, 128)], rows_t)

        def init_g(g, _):
            dv = dinv_t[pl.ds(r0 + g * L, L)]
            sv = dv * dv * cmask
            for i in range(L):
                s = sv[i]
                e = g * L + i
                for j in range(C // L):
                    rows_t[e, pl.ds(j * L, L)] = (
                        rows_t[e, pl.ds(j * L, L)] * s)
            return 0
        lax.fori_loop(0, 128 // L, init_g, 0)
        pltpu.sync_copy(rows_t, acc_sm.at[pl.ds(r0, 128)])
        return 0
    lax.fori_loop(0, NSL // 128, init_chunk, 0)
    plsc.subcore_barrier()

    ebase = cid * (E_ROWS // NC) + sid * EDGE_ROWS

    def edge_chunk(ci, _):
        r0 = ebase + ci * EPB
        pltpu.sync_copy(srcR.at[pl.ds(r0, EPB)], src_t)
        pltpu.sync_copy(dstR.at[pl.ds(r0, EPB)], dst_t)
        pltpu.sync_copy(ewR.at[pl.ds(r0, EPB)], ew_t)

        def group(j, _):
            pltpu.sync_copy(xw_hbm.at[src_t.at[j]], rows_t)
            for k in range(128 // L):
                sv = src_t[j, pl.ds(k * L, L)]
                dv = dst_t[j, pl.ds(k * L, L)]
                wv = ew_t[j, pl.ds(k * L, L)]
                nv = (plsc.load_gather(dinv_t, [sv])
                      * plsc.load_gather(dinv_t, [dv]) * wv)
                norm_t[pl.ds(k * L, L)] = nv

            def scale_g(g, _):
                sv = norm_t[pl.ds(g * L, L)]
                for i in range(L):
                    s = sv[i]
                    e = g * L + i
                    for j8 in range(C // L):
                        rows_t[e, pl.ds(j8 * L, L)] = (
                            rows_t[e, pl.ds(j8 * L, L)] * s)
                return 0
            lax.fori_loop(0, 128 // L, scale_g, 0)
            pltpu.sync_copy(rows_t, acc_sm.at[dst_t.at[j]], add=True)
            return 0
        lax.fori_loop(0, EPB, group, 0)
        return 0
    lax.fori_loop(0, EDGE_ROWS // EPB, edge_chunk, 0)
    plsc.subcore_barrier()

    pltpu.sync_copy(acc_sm.at[pl.ds(nbase, NSL)],
                    out_hbm.at[pl.ds(cid * NPAD + nbase, NSL)])


def _sc_edge(srcR, dstR, ewR, xw, dinv):
    mesh = plsc.VectorSubcoreMesh(core_axis_name="c", subcore_axis_name="s",
                                  num_cores=NC, num_subcores=NS)
    f = functools.partial(
        pl.kernel,
        out_type=jax.ShapeDtypeStruct((2 * NPAD, C), jnp.float32),
        mesh=mesh,
        compiler_params=pltpu.CompilerParams(needs_layout_passes=False),
        scratch_types=[
            pltpu.VMEM_SHARED((NPAD, C), jnp.float32),
            pltpu.VMEM((NPAD,), jnp.float32),
            pltpu.VMEM((EPB, 128), jnp.int32),
            pltpu.VMEM((EPB, 128), jnp.int32),
            pltpu.VMEM((EPB, 128), jnp.float32),
            pltpu.VMEM((128,), jnp.float32),
            pltpu.VMEM((128, C), jnp.float32),
        ],
    )(_sc_body)
    return f(srcR, dstR, ewR, xw, dinv)


def kernel(X, edge_index, edge_weight, W, W_ih, W_hh, b_ih, b_hh):
    del W_hh
    epad = E_ROWS * 128
    e = edge_index.shape[1]
    src = jnp.zeros((epad,), jnp.int32).at[:e].set(
        edge_index[0].astype(jnp.int32))
    dst = jnp.zeros((epad,), jnp.int32).at[:e].set(
        edge_index[1].astype(jnp.int32))
    ew = jnp.zeros((epad,), jnp.float32).at[:e].set(edge_weight)
    srcR = src.reshape(E_ROWS, 128)
    dstR = dst.reshape(E_ROWS, 128)
    ewR = ew.reshape(E_ROWS, 128)

    degp = _sc_deg(dstR, ewR)

    x_pad = jnp.zeros((NPAD, C), jnp.float32).at[:N].set(X)
    xw, dinv2d = _tc_prep(x_pad, W, W_ih,
                          b_ih.reshape(1, 4 * C), b_hh.reshape(1, 4 * C),
                          degp.reshape(2 * NPAD // C, C))

    partials = _sc_edge(srcR, dstR, ewR, xw, dinv2d.reshape(NPAD))
    return _tc_combine(partials)
